# baseline (speedup 1.0000x reference)
import functools

import jax
import jax.numpy as jnp
from jax import lax
from jax.experimental import pallas as pl
from jax.experimental.pallas import tpu as pltpu
from jax.experimental.pallas import tpu_sc as plsc

N_ROWS = 262144
N_COLS = 128
LANES = 16
GROUPS = N_COLS // LANES
N_WORKERS = 32
ROWS_PER_WORKER = N_ROWS // N_WORKERS
BLOCK_ROWS = 256
NB = ROWS_PER_WORKER // BLOCK_ROWS
NBUF = 2
UNROLL = 4

_MESH = plsc.VectorSubcoreMesh(core_axis_name="c", subcore_axis_name="s")


@functools.partial(
    pl.kernel,
    out_type=jax.ShapeDtypeStruct((N_WORKERS, N_COLS), jnp.float32),
    mesh=_MESH,
    scratch_types=[
        pltpu.VMEM((NBUF, BLOCK_ROWS, N_COLS), jnp.float32),
        pltpu.VMEM((N_COLS,), jnp.float32),
        pltpu.VMEM((N_COLS,), jnp.float32),
        pltpu.SemaphoreType.DMA,
        pltpu.SemaphoreType.DMA,
    ],
)
def _sc_count(tens_hbm, part_hbm, buf, prow, acc_s, sem0, sem1):
    sems = [sem0, sem1]
    wid = lax.axis_index("s") * 2 + lax.axis_index("c")
    base = wid * ROWS_PER_WORKER
    seed_row = jnp.maximum(base - 1, 0)
    pltpu.sync_copy(tens_hbm.at[seed_row], prow)

    def block_copy(blk, slot):
        return pltpu.make_async_copy(
            tens_hbm.at[pl.ds(base + blk * BLOCK_ROWS, BLOCK_ROWS), :],
            buf.at[slot],
            sems[slot],
        )

    for b in range(NBUF):
        block_copy(b, b).start()

    prev0 = tuple(
        jnp.where(prow[pl.ds(g * LANES, LANES)] > 0.0, 0.0, 1.0)
        for g in range(GROUPS)
    )
    acc0 = tuple(jnp.zeros((LANES,), jnp.float32) for _ in range(GROUPS))

    def outer(ci, carry):
        for b in range(NBUF):
            blk = ci * NBUF + b
            block_copy(blk, b).wait()

            def row_body(ri, c2, _b=b):
                pm = list(c2[:GROUPS])
                ac = list(c2[GROUPS:])
                r = ri * UNROLL
                for u in range(UNROLL):
                    for g in range(GROUPS):
                        cur = buf[_b, r + u, pl.ds(g * LANES, LANES)]
                        pos = jnp.where(cur > 0.0, 1.0, 0.0)
                        ac[g] = ac[g] + pos * pm[g]
                        pm[g] = 1.0 - pos
                return tuple(pm) + tuple(ac)

            carry = lax.fori_loop(0, BLOCK_ROWS // UNROLL, row_body, carry)

            nxt = blk + NBUF

            @pl.when(nxt < NB)
            def _():
                block_copy(nxt, b).start()

        return carry

    carry = lax.fori_loop(0, NB // NBUF, outer, prev0 + acc0)
    for g in range(GROUPS):
        acc_s[pl.ds(g * LANES, LANES)] = carry[GROUPS + g]
    pltpu.sync_copy(acc_s, part_hbm.at[wid])


def _sum_body(p_ref, o_ref):
    o_ref[...] = jnp.sum(p_ref[...], axis=0, keepdims=True)


def kernel(tens, values):
    del values
    parts = _sc_count(tens)
    out = pl.pallas_call(
        _sum_body,
        out_shape=jax.ShapeDtypeStruct((1, N_COLS), jnp.float32),
    )(parts)
    return out[0]


# speedup vs baseline: 1.7707x; 1.7707x over previous
import functools

import jax
import jax.numpy as jnp
from jax import lax
from jax.experimental import pallas as pl
from jax.experimental.pallas import tpu as pltpu
from jax.experimental.pallas import tpu_sc as plsc

N_ROWS = 262144
N_COLS = 128
LANES = 16
GROUPS = N_COLS // LANES
N_WORKERS = 32
ROWS_PER_WORKER = N_ROWS // N_WORKERS
BLOCK_ROWS = 256
NB = ROWS_PER_WORKER // BLOCK_ROWS
NBUF = 2
UNROLL = 4

_MESH = plsc.VectorSubcoreMesh(core_axis_name="c", subcore_axis_name="s")


@functools.partial(
    pl.kernel,
    out_type=jax.ShapeDtypeStruct((N_WORKERS, N_COLS), jnp.float32),
    mesh=_MESH,
    scratch_types=[
        pltpu.VMEM((NBUF, BLOCK_ROWS, N_COLS), jnp.float32),
        pltpu.VMEM((N_COLS,), jnp.float32),
        pltpu.VMEM((N_COLS,), jnp.float32),
        pltpu.SemaphoreType.DMA,
        pltpu.SemaphoreType.DMA,
    ],
)
def _sc_count(tens_hbm, part_hbm, buf, prow, acc_s, sem0, sem1):
    sems = [sem0, sem1]
    wid = lax.axis_index("s") * 2 + lax.axis_index("c")
    base = wid * ROWS_PER_WORKER
    seed_row = jnp.maximum(base - 1, 0)
    pltpu.sync_copy(tens_hbm.at[seed_row], prow)

    def block_copy(blk, slot):
        return pltpu.make_async_copy(
            tens_hbm.at[pl.ds(base + blk * BLOCK_ROWS, BLOCK_ROWS), :],
            buf.at[slot],
            sems[slot],
        )

    for b in range(NBUF):
        block_copy(b, b).start()

    prev0 = tuple(
        jnp.where(prow[pl.ds(g * LANES, LANES)] > 0.0, 0.0, 1.0)
        for g in range(GROUPS)
    )
    acc0 = tuple(jnp.zeros((LANES,), jnp.float32) for _ in range(GROUPS))

    def outer(ci, carry):
        for b in range(NBUF):
            blk = ci * NBUF + b
            block_copy(blk, b).wait()

            def row_body(r, c2, _b=b):
                pm = list(c2[:GROUPS])
                ac = list(c2[GROUPS:])
                for g in range(GROUPS):
                    cur = buf[_b, r, pl.ds(g * LANES, LANES)]
                    pos = jnp.where(cur > 0.0, 1.0, 0.0)
                    ac[g] = ac[g] + pos * pm[g]
                    pm[g] = 1.0 - pos
                return tuple(pm) + tuple(ac)

            carry = plsc.parallel_loop(
                0, BLOCK_ROWS, step=1, unroll=UNROLL, carry=carry
            )(row_body)

            nxt = blk + NBUF

            @pl.when(nxt < NB)
            def _():
                block_copy(nxt, b).start()

        return carry

    carry = lax.fori_loop(0, NB // NBUF, outer, prev0 + acc0)
    for g in range(GROUPS):
        acc_s[pl.ds(g * LANES, LANES)] = carry[GROUPS + g]
    pltpu.sync_copy(acc_s, part_hbm.at[wid])


def _sum_body(p_ref, o_ref):
    o_ref[...] = jnp.sum(p_ref[...], axis=0, keepdims=True)


def kernel(tens, values):
    del values
    parts = _sc_count(tens)
    out = pl.pallas_call(
        _sum_body,
        out_shape=jax.ShapeDtypeStruct((1, N_COLS), jnp.float32),
    )(parts)
    return out[0]
